# baseline (speedup 1.0000x reference)
import functools

import jax
import jax.numpy as jnp
from jax import lax
from jax.experimental import pallas as pl
from jax.experimental.pallas import tpu as pltpu
from jax.experimental.pallas import tpu_sc as plsc

_INFO = plsc.get_sparse_core_info()
_NC = _INFO.num_cores
_NS = _INFO.num_subcores
_NW = _NC * _NS


@functools.lru_cache(maxsize=None)
def _make_gather(V, D, B):
    assert D % 8 == 0 and B % (_NW * 128) == 0
    b_per_w = B // _NW
    n_bt = b_per_w // 128
    n_rt = D // 8
    n_tiles = n_bt * n_rt
    n_f16 = b_per_w // 16

    mesh = plsc.VectorSubcoreMesh(core_axis_name="c", subcore_axis_name="s")

    @functools.partial(
        pl.kernel,
        mesh=mesh,
        out_type=jax.ShapeDtypeStruct((n_rt, B // 128, 8, 128), jnp.float32),
        scratch_types=[
            pltpu.VMEM((b_per_w,), jnp.int32),
            pltpu.VMEM((b_per_w, 128), jnp.float32),
            pltpu.VMEM((n_rt, n_bt, 8, 128), jnp.float32),
            [pltpu.SemaphoreType.DMA] * n_bt,
        ],
        compiler_params=pltpu.CompilerParams(
            use_tc_tiling_on_sc=True, needs_layout_passes=False
        ),
    )
    def gather_kernel(table_hbm, idx_hbm, out_hbm, idx_v, rows_v, t4_v, sems):
        wid = lax.axis_index("s") * _NC + lax.axis_index("c")
        base = wid * b_per_w
        cbase = wid * n_bt
        pltpu.sync_copy(idx_hbm.at[pl.ds(base, b_per_w)], idx_v)
        iota = lax.iota(jnp.int32, 16)

        copies = [
            pltpu.async_copy(
                table_hbm.at[idx_v.at[pl.ds(jb * 128, 128)]],
                rows_v.at[pl.ds(jb * 128, 128)],
                sems[jb],
            )
            for jb in range(n_bt)
        ]

        for jb in range(n_bt):
            copies[jb].wait()

            def emit_group(rr, _, jb=jb):
                for dr in range(8):
                    cidx = iota * 0 + (rr * 8 + dr)
                    for o0 in range(0, 128, 16):
                        ridx = jb * 128 + o0 + iota
                        vals = plsc.load_gather(rows_v, [ridx, cidx])
                        t4_v[rr, jb, dr, pl.ds(o0, 16)] = vals
                return 0

            lax.fori_loop(0, n_rt, emit_group, 0, unroll=False)

        pltpu.sync_copy(t4_v, out_hbm.at[:, pl.ds(cbase, n_bt)])

    return gather_kernel


def kernel(winding_indices, weight):
    B = winding_indices.shape[0]
    V, D = weight.shape
    fn = _make_gather(V, D, B)
    w128 = jnp.pad(weight, ((0, 0), (0, 128 - D)))
    o4 = fn(w128, winding_indices)
    return o4.transpose(1, 3, 0, 2).reshape(B, D)


# speedup vs baseline: 1.0022x; 1.0022x over previous
import functools

import jax
import jax.numpy as jnp
from jax import lax
from jax.experimental import pallas as pl
from jax.experimental.pallas import tpu as pltpu
from jax.experimental.pallas import tpu_sc as plsc

_INFO = plsc.get_sparse_core_info()
_NC = _INFO.num_cores
_NS = _INFO.num_subcores
_NW = _NC * _NS


@functools.lru_cache(maxsize=None)
def _make_gather(V, D, B):
    assert D % 8 == 0 and B % (_NW * 128) == 0
    b_per_w = B // _NW
    n_bt = b_per_w // 128
    n_rt = D // 8
    n_tiles = n_bt * n_rt
    n_f16 = b_per_w // 16

    mesh = plsc.VectorSubcoreMesh(core_axis_name="c", subcore_axis_name="s")

    @functools.partial(
        pl.kernel,
        mesh=mesh,
        out_type=jax.ShapeDtypeStruct((n_rt, B // 128, 8, 128), jnp.float32),
        scratch_types=[
            pltpu.VMEM((b_per_w,), jnp.int32),
            pltpu.VMEM((b_per_w, 128), jnp.float32),
            pltpu.VMEM((n_rt, n_bt, 8, 128), jnp.float32),
            [pltpu.SemaphoreType.DMA] * n_bt,
        ],
        compiler_params=pltpu.CompilerParams(
            use_tc_tiling_on_sc=True, needs_layout_passes=False
        ),
    )
    def gather_kernel(table_hbm, idx_hbm, out_hbm, idx_v, rows_v, t4_v, sems):
        wid = lax.axis_index("s") * _NC + lax.axis_index("c")
        base = wid * b_per_w
        cbase = wid * n_bt
        pltpu.sync_copy(idx_hbm.at[pl.ds(base, b_per_w)], idx_v)
        iota = lax.iota(jnp.int32, 16)

        for jb in range(n_bt):
            def fetch16(i, _, jb=jb):
                k = jb * 128 + i * 16
                vec = idx_v[pl.ds(k, 16)]
                pltpu.async_copy(
                    table_hbm.at[vec],
                    rows_v.at[pl.ds(k, 16)],
                    sems[jb],
                )
                return 0

            lax.fori_loop(0, 8, fetch16, 0, unroll=False)

        for jb in range(n_bt):
            pltpu.make_async_copy(
                table_hbm.at[pl.ds(0, 128)],
                rows_v.at[pl.ds(jb * 128, 128)],
                sems[jb],
            ).wait()

            def emit_group(rr, _, jb=jb):
                for dr in range(8):
                    cidx = iota * 0 + (rr * 8 + dr)
                    for o0 in range(0, 128, 16):
                        ridx = jb * 128 + o0 + iota
                        vals = plsc.load_gather(rows_v, [ridx, cidx])
                        t4_v[rr, jb, dr, pl.ds(o0, 16)] = vals
                return 0

            lax.fori_loop(0, n_rt, emit_group, 0, unroll=False)

        pltpu.sync_copy(t4_v, out_hbm.at[:, pl.ds(cbase, n_bt)])

    return gather_kernel


def kernel(winding_indices, weight):
    B = winding_indices.shape[0]
    V, D = weight.shape
    fn = _make_gather(V, D, B)
    w128 = jnp.pad(weight, ((0, 0), (0, 128 - D)))
    o4 = fn(w128, winding_indices)
    return o4.transpose(1, 3, 0, 2).reshape(B, D)


# speedup vs baseline: 1.3423x; 1.3394x over previous
import functools

import jax
import jax.numpy as jnp
from jax import lax
from jax.experimental import pallas as pl
from jax.experimental.pallas import tpu as pltpu
from jax.experimental.pallas import tpu_sc as plsc

_INFO = plsc.get_sparse_core_info()
_NC = _INFO.num_cores
_NS = _INFO.num_subcores
_NW = _NC * _NS


@functools.lru_cache(maxsize=None)
def _make_gather(V, D, B):
    assert D % 8 == 0 and B % (_NW * 128) == 0
    b_per_w = B // _NW
    n_bt = b_per_w // 128
    n_rt = D // 8
    n_tiles = n_bt * n_rt
    n_f16 = b_per_w // 16

    mesh = plsc.VectorSubcoreMesh(core_axis_name="c", subcore_axis_name="s")

    @functools.partial(
        pl.kernel,
        mesh=mesh,
        out_type=jax.ShapeDtypeStruct((n_rt, B // 128, 8, 128), jnp.float32),
        scratch_types=[
            pltpu.VMEM((b_per_w,), jnp.int32),
            pltpu.VMEM((b_per_w, D), jnp.float32),
            pltpu.VMEM((2, 8, 128), jnp.float32),
            pltpu.SemaphoreType.DMA,
            pltpu.SemaphoreType.DMA,
        ],
        compiler_params=pltpu.CompilerParams(
            use_tc_tiling_on_sc=True, needs_layout_passes=False
        ),
    )
    def gather_kernel(table_hbm, idx_hbm, out_hbm, idx_v, rows_v, t4_v, sem_in, sem_out):
        wid = lax.axis_index("s") * _NC + lax.axis_index("c")
        base = wid * b_per_w
        cbase = wid * n_bt
        pltpu.sync_copy(idx_hbm.at[pl.ds(base, b_per_w)], idx_v)
        iota = lax.iota(jnp.int32, 16)

        def fetch16(i, _):
            vec = idx_v[pl.ds(i * 16, 16)]
            for j in range(16):
                v = vec[j]
                pltpu.async_copy(
                    table_hbm.at[pl.ds(v, 1), :],
                    rows_v.at[pl.ds(i * 16 + j, 1), :],
                    sem_in,
                )
            return 0

        lax.fori_loop(0, n_f16, fetch16, 0, unroll=False)
        pltpu.make_async_copy(
            table_hbm.at[pl.ds(0, b_per_w), :],
            rows_v,
            sem_in,
        ).wait()

        def emit_tile(t, _):
            jb = t // n_rt
            rr = t % n_rt
            buf = t % 2

            @pl.when(t >= 2)
            def _():
                pltpu.make_async_copy(
                    t4_v.at[0], out_hbm.at[0, 0], sem_out
                ).wait()

            for dr in range(8):
                cidx = iota * 0 + (rr * 8 + dr)
                for o0 in range(0, 128, 16):
                    ridx = jb * 128 + o0 + iota
                    vals = plsc.load_gather(rows_v, [ridx, cidx])
                    t4_v[buf, dr, pl.ds(o0, 16)] = vals

            pltpu.async_copy(t4_v.at[buf], out_hbm.at[rr, cbase + jb], sem_out)
            return 0

        lax.fori_loop(0, n_tiles, emit_tile, 0, unroll=False)
        pltpu.make_async_copy(t4_v.at[0], out_hbm.at[0, 0], sem_out).wait()
        pltpu.make_async_copy(t4_v.at[0], out_hbm.at[0, 0], sem_out).wait()

    return gather_kernel


def kernel(winding_indices, weight):
    B = winding_indices.shape[0]
    V, D = weight.shape
    fn = _make_gather(V, D, B)
    o4 = fn(weight, winding_indices)
    return o4.transpose(1, 3, 0, 2).reshape(B, D)
